# baseline (speedup 1.0000x reference)
import jax
import jax.numpy as jnp
from jax.experimental import pallas as pl
from jax.experimental.pallas import tpu as pltpu

Q, D, K = 3136, 1536, 16384
BQ, BK = 256, 2048
QP = 3328
QBLOCKS = QP // BQ
KBLOCKS = K // BK
NSTEPS = QBLOCKS * KBLOCKS
TOPK = 3


def _extract_merge(st, base, qs, sv_ref, si_ref):
    iota = jax.lax.broadcasted_iota(jnp.int32, (BK, BQ), 0)
    bvals, bidxs = [], []
    scur = st
    for t in range(TOPK):
        m = jnp.min(scur, axis=0, keepdims=True)
        loc = jnp.min(jnp.where(scur == m, iota, BK), axis=0,
                      keepdims=True)
        bvals.append(m)
        bidxs.append(loc + base)
        if t < TOPK - 1:
            scur = jnp.where(iota == loc, jnp.inf, scur)

    v6 = jnp.concatenate([sv_ref[0:TOPK, qs]] + bvals, axis=0)
    i6 = jnp.concatenate([si_ref[0:TOPK, qs]] + bidxs, axis=0)
    iota6 = jax.lax.broadcasted_iota(jnp.int32, (2 * TOPK, BQ), 0)
    outv, outi = [], []
    for t in range(TOPK):
        m = jnp.min(v6, axis=0, keepdims=True)
        pos = jnp.min(jnp.where(v6 == m, iota6, 2 * TOPK), axis=0,
                      keepdims=True)
        sel = iota6 == pos
        outv.append(m)
        outi.append(jnp.sum(jnp.where(sel, i6, 0), axis=0, keepdims=True))
        if t < TOPK - 1:
            v6 = jnp.where(sel, jnp.inf, v6)
    sv_ref[0:TOPK, qs] = jnp.concatenate(outv, axis=0)
    si_ref[0:TOPK, qs] = jnp.concatenate(outi, axis=0)


def _body(x_ref, yt_ref, val_ref, idx_ref, sv_ref, si_ref, y2_ref,
          st_ref, x2_ref):
    kj = pl.program_id(0)
    qi = pl.program_id(1)
    n = kj * QBLOCKS + qi
    qs = pl.ds(qi * BQ, BQ)

    @pl.when(qi == 0)
    def _cache_y2():
        yt = yt_ref[...]
        y2_ref[0:1, :] = jnp.sum(yt * yt, axis=0, keepdims=True)

    @pl.when(kj == 0)
    def _init():
        sv_ref[:, qs] = jnp.full((8, BQ), jnp.inf, jnp.float32)
        si_ref[:, qs] = jnp.zeros((8, BQ), jnp.int32)
        x2_ref[0:1, qs] = jnp.sum(x_ref[...] ** 2, axis=1, keepdims=True).T

    @pl.when(n > 0)
    def _consume_prev():
        pn = n - 1
        pkj = pn // QBLOCKS
        pqi = pn % QBLOCKS
        _extract_merge(st_ref[...], pkj * BK, pl.ds(pqi * BQ, BQ),
                       sv_ref, si_ref)

    d = jax.lax.dot_general(
        x_ref[...], yt_ref[...], (((1,), (0,)), ((), ())),
        preferred_element_type=jnp.float32,
    )
    s = y2_ref[0:1, :] - 2.0 * d
    st_ref[...] = s.T

    @pl.when(n == NSTEPS - 1)
    def _finish():
        _extract_merge(st_ref[...], kj * BK, qs, sv_ref, si_ref)
        val_ref[...] = jnp.sqrt(
            jnp.maximum(sv_ref[0:TOPK, :] + x2_ref[0:1, :], 1e-12))
        idx_ref[...] = si_ref[0:TOPK, :]


def kernel(features, memory_bank):
    xp = jnp.pad(features, ((0, QP - Q), (0, 0)))
    mb_t = memory_bank.T
    vals_t, idxs_t = pl.pallas_call(
        _body,
        grid=(KBLOCKS, QBLOCKS),
        in_specs=[
            pl.BlockSpec((BQ, D), lambda kj, qi: (qi, 0)),
            pl.BlockSpec((D, BK), lambda kj, qi: (0, kj)),
        ],
        out_specs=[
            pl.BlockSpec((TOPK, QP), lambda kj, qi: (0, 0)),
            pl.BlockSpec((TOPK, QP), lambda kj, qi: (0, 0)),
        ],
        out_shape=[
            jax.ShapeDtypeStruct((TOPK, QP), jnp.float32),
            jax.ShapeDtypeStruct((TOPK, QP), jnp.int32),
        ],
        scratch_shapes=[
            pltpu.VMEM((8, QP), jnp.float32),
            pltpu.VMEM((8, QP), jnp.int32),
            pltpu.VMEM((8, BK), jnp.float32),
            pltpu.VMEM((BK, BQ), jnp.float32),
            pltpu.VMEM((8, QP), jnp.float32),
        ],
        compiler_params=pltpu.CompilerParams(
            dimension_semantics=("arbitrary", "arbitrary"),
        ),
    )(xp, mb_t)
    return vals_t[:, :Q].T, idxs_t[:, :Q].T


# speedup vs baseline: 1.0552x; 1.0552x over previous
import jax
import jax.numpy as jnp
from jax.experimental import pallas as pl
from jax.experimental.pallas import tpu as pltpu

Q, D, K = 3136, 1536, 16384
BQ, BK = 256, 2048
QP = 3328
QBLOCKS = QP // BQ
KBLOCKS = K // BK
NSTEPS = QBLOCKS * KBLOCKS
TOPK = 3


def _extract_merge(st, base, qs, sv_ref, si_ref):
    iota = jax.lax.broadcasted_iota(jnp.int32, (BK, BQ), 0)
    bvals, bidxs = [], []
    scur = st
    for t in range(TOPK):
        m = jnp.min(scur, axis=0, keepdims=True)
        loc = jnp.min(jnp.where(scur == m, iota, BK), axis=0,
                      keepdims=True)
        bvals.append(m)
        bidxs.append(loc + base)
        if t < TOPK - 1:
            scur = jnp.where(iota == loc, jnp.inf, scur)

    v6 = jnp.concatenate([sv_ref[0:TOPK, qs]] + bvals, axis=0)
    i6 = jnp.concatenate([si_ref[0:TOPK, qs]] + bidxs, axis=0)
    iota6 = jax.lax.broadcasted_iota(jnp.int32, (2 * TOPK, BQ), 0)
    outv, outi = [], []
    for t in range(TOPK):
        m = jnp.min(v6, axis=0, keepdims=True)
        pos = jnp.min(jnp.where(v6 == m, iota6, 2 * TOPK), axis=0,
                      keepdims=True)
        sel = iota6 == pos
        outv.append(m)
        outi.append(jnp.sum(jnp.where(sel, i6, 0), axis=0, keepdims=True))
        if t < TOPK - 1:
            v6 = jnp.where(sel, jnp.inf, v6)
    sv_ref[0:TOPK, qs] = jnp.concatenate(outv, axis=0)
    si_ref[0:TOPK, qs] = jnp.concatenate(outi, axis=0)


def _body(x_ref, yt_ref, val_ref, idx_ref, sv_ref, si_ref, y2_ref,
          st_ref, x2_ref):
    kj = pl.program_id(0)
    qi = pl.program_id(1)
    n = kj * QBLOCKS + qi
    qs = pl.ds(qi * BQ, BQ)

    @pl.when(qi == 0)
    def _cache_y2():
        yt = yt_ref[...]
        y2_ref[0:1, :] = jnp.sum(yt * yt, axis=0, keepdims=True)

    @pl.when(kj == 0)
    def _init():
        sv_ref[:, qs] = jnp.full((8, BQ), jnp.inf, jnp.float32)
        si_ref[:, qs] = jnp.zeros((8, BQ), jnp.int32)
        x2_ref[0:1, qs] = jnp.sum(x_ref[...] ** 2, axis=1, keepdims=True).T

    @pl.when(n == 0)
    def _prefill():
        st_ref[...] = jnp.full((2 * BK, BQ), jnp.inf, jnp.float32)

    pn = n - 1
    pkj = pn // QBLOCKS
    pqi = pn - pkj * QBLOCKS
    woff = (n % 2) * BK
    pst = st_ref[pl.ds(BK - woff, BK), :]
    _extract_merge(pst, pkj * BK, pl.ds(pqi * BQ, BQ), sv_ref, si_ref)

    d = jax.lax.dot_general(
        x_ref[...], yt_ref[...], (((1,), (0,)), ((), ())),
        preferred_element_type=jnp.float32,
    )
    s = y2_ref[0:1, :] - 2.0 * d
    st = s.T
    st_ref[pl.ds(woff, BK), :] = st

    @pl.when(n == NSTEPS - 1)
    def _finish():
        _extract_merge(st, kj * BK, qs, sv_ref, si_ref)
        val_ref[...] = jnp.sqrt(
            jnp.maximum(sv_ref[0:TOPK, :] + x2_ref[0:1, :], 1e-12))
        idx_ref[...] = si_ref[0:TOPK, :]


def kernel(features, memory_bank):
    xp = jnp.pad(features, ((0, QP - Q), (0, 0)))
    mb_t = memory_bank.T
    vals_t, idxs_t = pl.pallas_call(
        _body,
        grid=(KBLOCKS, QBLOCKS),
        in_specs=[
            pl.BlockSpec((BQ, D), lambda kj, qi: (qi, 0)),
            pl.BlockSpec((D, BK), lambda kj, qi: (0, kj)),
        ],
        out_specs=[
            pl.BlockSpec((TOPK, QP), lambda kj, qi: (0, 0)),
            pl.BlockSpec((TOPK, QP), lambda kj, qi: (0, 0)),
        ],
        out_shape=[
            jax.ShapeDtypeStruct((TOPK, QP), jnp.float32),
            jax.ShapeDtypeStruct((TOPK, QP), jnp.int32),
        ],
        scratch_shapes=[
            pltpu.VMEM((8, QP), jnp.float32),
            pltpu.VMEM((8, QP), jnp.int32),
            pltpu.VMEM((8, BK), jnp.float32),
            pltpu.VMEM((2 * BK, BQ), jnp.float32),
            pltpu.VMEM((8, QP), jnp.float32),
        ],
        compiler_params=pltpu.CompilerParams(
            dimension_semantics=("arbitrary", "arbitrary"),
        ),
    )(xp, mb_t)
    return vals_t[:, :Q].T, idxs_t[:, :Q].T


# speedup vs baseline: 1.1030x; 1.0453x over previous
import jax
import jax.numpy as jnp
from jax.experimental import pallas as pl
from jax.experimental.pallas import tpu as pltpu

Q, D, K = 3136, 1536, 16384
BQ, BK = 256, 2048
QP = 3328
QBLOCKS = QP // BQ
KBLOCKS = K // BK
NSTEPS = QBLOCKS * KBLOCKS
TOPK = 3


def _extract_merge(st, base, qs, sv_ref, si_ref):
    iota = jax.lax.broadcasted_iota(jnp.int32, (BK, BQ), 0)
    bvals, bidxs = [], []
    scur = st
    for t in range(TOPK):
        m = jnp.min(scur, axis=0, keepdims=True)
        loc = jnp.argmin(scur, axis=0).astype(jnp.int32)[None, :]
        bvals.append(m)
        bidxs.append(loc + base)
        if t < TOPK - 1:
            scur = jnp.where(iota == loc, jnp.inf, scur)

    v6 = jnp.concatenate([sv_ref[0:TOPK, qs]] + bvals, axis=0)
    i6 = jnp.concatenate([si_ref[0:TOPK, qs]] + bidxs, axis=0)
    iota6 = jax.lax.broadcasted_iota(jnp.int32, (2 * TOPK, BQ), 0)
    outv, outi = [], []
    for t in range(TOPK):
        m = jnp.min(v6, axis=0, keepdims=True)
        pos = jnp.min(jnp.where(v6 == m, iota6, 2 * TOPK), axis=0,
                      keepdims=True)
        sel = iota6 == pos
        outv.append(m)
        outi.append(jnp.sum(jnp.where(sel, i6, 0), axis=0, keepdims=True))
        if t < TOPK - 1:
            v6 = jnp.where(sel, jnp.inf, v6)
    sv_ref[0:TOPK, qs] = jnp.concatenate(outv, axis=0)
    si_ref[0:TOPK, qs] = jnp.concatenate(outi, axis=0)


def _body(x_ref, yt_ref, val_ref, idx_ref, sv_ref, si_ref, y2_ref,
          st_ref, x2_ref):
    kj = pl.program_id(0)
    qi = pl.program_id(1)
    n = kj * QBLOCKS + qi
    qs = pl.ds(qi * BQ, BQ)

    @pl.when(qi == 0)
    def _cache_y2():
        yt = yt_ref[...]
        y2_ref[0:1, :] = jnp.sum(yt * yt, axis=0, keepdims=True)

    @pl.when(kj == 0)
    def _init():
        sv_ref[:, qs] = jnp.full((8, BQ), jnp.inf, jnp.float32)
        si_ref[:, qs] = jnp.zeros((8, BQ), jnp.int32)
        x2_ref[0:1, qs] = jnp.sum(x_ref[...] ** 2, axis=1, keepdims=True).T

    @pl.when(n == 0)
    def _prefill():
        st_ref[...] = jnp.full((2 * BK, BQ), jnp.inf, jnp.float32)

    pn = n - 1
    pkj = pn // QBLOCKS
    pqi = pn - pkj * QBLOCKS
    woff = (n % 2) * BK
    pst = st_ref[pl.ds(BK - woff, BK), :]
    _extract_merge(pst, pkj * BK, pl.ds(pqi * BQ, BQ), sv_ref, si_ref)

    d = jax.lax.dot_general(
        x_ref[...], yt_ref[...], (((1,), (0,)), ((), ())),
        preferred_element_type=jnp.float32,
    )
    s = y2_ref[0:1, :] - 2.0 * d
    st = s.T
    st_ref[pl.ds(woff, BK), :] = st

    @pl.when(n == NSTEPS - 1)
    def _finish():
        _extract_merge(st, kj * BK, qs, sv_ref, si_ref)
        val_ref[...] = jnp.sqrt(
            jnp.maximum(sv_ref[0:TOPK, :] + x2_ref[0:1, :], 1e-12))
        idx_ref[...] = si_ref[0:TOPK, :]


def kernel(features, memory_bank):
    xp = jnp.pad(features, ((0, QP - Q), (0, 0)))
    mb_t = memory_bank.T
    vals_t, idxs_t = pl.pallas_call(
        _body,
        grid=(KBLOCKS, QBLOCKS),
        in_specs=[
            pl.BlockSpec((BQ, D), lambda kj, qi: (qi, 0)),
            pl.BlockSpec((D, BK), lambda kj, qi: (0, kj)),
        ],
        out_specs=[
            pl.BlockSpec((TOPK, QP), lambda kj, qi: (0, 0)),
            pl.BlockSpec((TOPK, QP), lambda kj, qi: (0, 0)),
        ],
        out_shape=[
            jax.ShapeDtypeStruct((TOPK, QP), jnp.float32),
            jax.ShapeDtypeStruct((TOPK, QP), jnp.int32),
        ],
        scratch_shapes=[
            pltpu.VMEM((8, QP), jnp.float32),
            pltpu.VMEM((8, QP), jnp.int32),
            pltpu.VMEM((8, BK), jnp.float32),
            pltpu.VMEM((2 * BK, BQ), jnp.float32),
            pltpu.VMEM((8, QP), jnp.float32),
        ],
        compiler_params=pltpu.CompilerParams(
            dimension_semantics=("arbitrary", "arbitrary"),
        ),
    )(xp, mb_t)
    return vals_t[:, :Q].T, idxs_t[:, :Q].T
